# baseline (speedup 1.0000x reference)
import jax
import jax.numpy as jnp
from jax.experimental import pallas as pl
from jax.experimental.pallas import tpu as pltpu

_N = 10000
_F_IN = 128
_F_OUT = 64
_BLOCK = 10000


def _gru_kernel(x_ref, ht_ref, wz_ref, bz_ref, wr_ref, br_ref, wh_ref, bh_ref,
                lw_ref, lb_ref, out_ref, hnew_ref):
    x = x_ref[...]
    h = ht_ref[...]
    wz = wz_ref[0, 0] + wz_ref[1, 0]
    wr = wr_ref[0, 0] + wr_ref[1, 0]
    wh = wh_ref[0, 0] + wh_ref[1, 0]
    bz = bz_ref[...][:, None]
    br = br_ref[...][:, None]
    bh = bh_ref[...][:, None]

    def mm(a, b):
        return jax.lax.dot_general(a, b, (((1,), (0,)), ((), ())),
                                   preferred_element_type=jnp.float32)

    def mmx(a, xb):
        return jax.lax.dot_general(a, xb, (((1,), (1,)), ((), ())),
                                   preferred_element_type=jnp.float32)

    z = jax.nn.sigmoid(mmx(wz[:, :_F_IN], x) + mm(wz[:, _F_IN:], h) + bz)
    r = jax.nn.sigmoid(mmx(wr[:, :_F_IN], x) + mm(wr[:, _F_IN:], h) + br)
    ht = jnp.tanh(mmx(wh[:, :_F_IN], x) + mm(wh[:, _F_IN:], h * r) + bh)
    h_new = z * h + (1.0 - z) * ht
    hnew_ref[...] = h_new
    relu_h = jnp.maximum(h_new, 0.0)
    out_ref[...] = mm(lw_ref[...], relu_h) + lb_ref[...][:, None]


def kernel(x, edge_index, edge_weight, memory, W_z, b_z, W_r, b_r, W_h, b_h,
           lin_w, lin_b):
    del edge_index, edge_weight
    mem_t = memory.T
    wz_t = W_z.transpose(0, 1, 3, 2)
    wr_t = W_r.transpose(0, 1, 3, 2)
    wh_t = W_h.transpose(0, 1, 3, 2)

    grid = pl.cdiv(_N, _BLOCK)
    col_spec = pl.BlockSpec((_F_OUT, _BLOCK), lambda i: (0, i))
    full = lambda shape: pl.BlockSpec(shape, lambda i: (0,) * len(shape))

    out_t, hnew_t = pl.pallas_call(
        _gru_kernel,
        grid=(grid,),
        in_specs=[
            pl.BlockSpec((_BLOCK, _F_IN), lambda i: (i, 0)),
            col_spec,
            full((2, 1, _F_OUT, _F_IN + _F_OUT)),
            full((_F_OUT,)),
            full((2, 1, _F_OUT, _F_IN + _F_OUT)),
            full((_F_OUT,)),
            full((2, 1, _F_OUT, _F_IN + _F_OUT)),
            full((_F_OUT,)),
            full((1, _F_OUT)),
            full((1,)),
        ],
        out_specs=[
            pl.BlockSpec((1, _BLOCK), lambda i: (0, i)),
            col_spec,
        ],
        out_shape=[
            jax.ShapeDtypeStruct((1, _N), jnp.float32),
            jax.ShapeDtypeStruct((_F_OUT, _N), jnp.float32),
        ],
        compiler_params=pltpu.CompilerParams(
            dimension_semantics=("arbitrary",),
        ),
    )(x, mem_t, wz_t, b_z, wr_t, b_r, wh_t, b_h, lin_w, lin_b)
    return (out_t.T, hnew_t.T)


# speedup vs baseline: 1.0573x; 1.0573x over previous
import jax
import jax.numpy as jnp
from jax.experimental import pallas as pl
from jax.experimental.pallas import tpu as pltpu

_N = 10000
_F_IN = 128
_F_OUT = 64
_BLOCK = 5120


def _gru_kernel(x_ref, ht_ref, wz_ref, bz_ref, wr_ref, br_ref, wh_ref, bh_ref,
                lw_ref, lb_ref, out_ref, hnew_ref):
    x = x_ref[...]
    h = ht_ref[...]
    wz = wz_ref[0, 0] + wz_ref[1, 0]
    wr = wr_ref[0, 0] + wr_ref[1, 0]
    wh = wh_ref[0, 0] + wh_ref[1, 0]
    bz = bz_ref[...][:, None]
    br = br_ref[...][:, None]
    bh = bh_ref[...][:, None]

    def mm(a, b):
        return jax.lax.dot_general(a, b, (((1,), (0,)), ((), ())),
                                   preferred_element_type=jnp.float32)

    def mmx(a, xb):
        return jax.lax.dot_general(a, xb, (((1,), (1,)), ((), ())),
                                   preferred_element_type=jnp.float32)

    z = jax.nn.sigmoid(mmx(wz[:, :_F_IN], x) + mm(wz[:, _F_IN:], h) + bz)
    r = jax.nn.sigmoid(mmx(wr[:, :_F_IN], x) + mm(wr[:, _F_IN:], h) + br)
    ht = jnp.tanh(mmx(wh[:, :_F_IN], x) + mm(wh[:, _F_IN:], h * r) + bh)
    h_new = z * h + (1.0 - z) * ht
    hnew_ref[...] = h_new
    relu_h = jnp.maximum(h_new, 0.0)
    out_ref[...] = mm(lw_ref[...], relu_h) + lb_ref[...][:, None]


def kernel(x, edge_index, edge_weight, memory, W_z, b_z, W_r, b_r, W_h, b_h,
           lin_w, lin_b):
    del edge_index, edge_weight
    mem_t = memory.T
    wz_t = W_z.transpose(0, 1, 3, 2)
    wr_t = W_r.transpose(0, 1, 3, 2)
    wh_t = W_h.transpose(0, 1, 3, 2)

    grid = pl.cdiv(_N, _BLOCK)
    col_spec = pl.BlockSpec((_F_OUT, _BLOCK), lambda i: (0, i))
    full = lambda shape: pl.BlockSpec(shape, lambda i: (0,) * len(shape))

    out_t, hnew_t = pl.pallas_call(
        _gru_kernel,
        grid=(grid,),
        in_specs=[
            pl.BlockSpec((_BLOCK, _F_IN), lambda i: (i, 0)),
            col_spec,
            full((2, 1, _F_OUT, _F_IN + _F_OUT)),
            full((_F_OUT,)),
            full((2, 1, _F_OUT, _F_IN + _F_OUT)),
            full((_F_OUT,)),
            full((2, 1, _F_OUT, _F_IN + _F_OUT)),
            full((_F_OUT,)),
            full((1, _F_OUT)),
            full((1,)),
        ],
        out_specs=[
            pl.BlockSpec((1, _BLOCK), lambda i: (0, i)),
            col_spec,
        ],
        out_shape=[
            jax.ShapeDtypeStruct((1, _N), jnp.float32),
            jax.ShapeDtypeStruct((_F_OUT, _N), jnp.float32),
        ],
        compiler_params=pltpu.CompilerParams(
            dimension_semantics=("arbitrary",),
        ),
    )(x, mem_t, wz_t, b_z, wr_t, b_r, wh_t, b_h, lin_w, lin_b)
    return (out_t.T, hnew_t.T)
